# Initial kernel scaffold; baseline (speedup 1.0000x reference)
import jax
import jax.numpy as jnp
from jax.experimental import pallas as pl


def kernel(bank, loc_map, labels, sel_idx):
    raise NotImplementedError("write your pallas kernel here")


# speedup vs baseline: 2.3434x; 2.3434x over previous
import functools

import jax
import jax.numpy as jnp
from jax import lax
from jax.experimental import pallas as pl
from jax.experimental.pallas import tpu as pltpu
from jax.experimental.pallas import tpu_sc as plsc

B, L, T = 8, 80, 256
C, NQ, NS, D = 365, 100, 4, 256
NQSEL = 5
RD = NS * D
NROWS = B * L * NQSEL
NW = 32
BPW_HI = 104
BPW_LO = 96
SPLIT = 16 * BPW_HI
REP = NQSEL * NS


def _sc_gather_rows(bank_flat, idx1d):
    mesh = plsc.VectorSubcoreMesh(core_axis_name="c", subcore_axis_name="s")

    @functools.partial(
        pl.kernel,
        mesh=mesh,
        out_type=jax.ShapeDtypeStruct((NROWS, RD), jnp.float32),
        scratch_types=[
            pltpu.VMEM((BPW_HI,), jnp.int32),
            pltpu.VMEM((BPW_HI, RD), jnp.float32),
            pltpu.SemaphoreType.DMA,
        ],
    )
    def k(table_hbm, idx_hbm, out_hbm, idx_v, rows_v, sem):
        wid = lax.axis_index("s") * 2 + lax.axis_index("c")
        pltpu.sync_copy(idx_hbm.at[pl.ds(pl.multiple_of(wid * BPW_HI, 8),
                                         BPW_HI)], idx_v)
        pltpu.async_copy(table_hbm.at[idx_v], rows_v, sem).wait()

        @pl.when(wid < 16)
        def _hi():
            base = pl.multiple_of(wid * BPW_HI, 8)
            pltpu.sync_copy(rows_v, out_hbm.at[pl.ds(base, BPW_HI)])

        @pl.when(wid >= 16)
        def _lo():
            base = pl.multiple_of(SPLIT + (wid - 16) * BPW_LO, 8)
            pltpu.sync_copy(rows_v.at[pl.ds(0, BPW_LO)],
                            out_hbm.at[pl.ds(base, BPW_LO)])

    return k(bank_flat, idx1d)


def _tc_mask_body(loc_ref, out_ref):
    x = loc_ref[...]
    y = (x != 0.0).astype(jnp.float32)
    out_ref[...] = jnp.broadcast_to(y, out_ref.shape)


def _tc_mask(loc3):
    blk = 16
    grid = (B * L // blk,)
    return pl.pallas_call(
        _tc_mask_body,
        grid=grid,
        in_specs=[pl.BlockSpec((blk, 1, T), lambda i: (i, 0, 0))],
        out_specs=pl.BlockSpec((blk, REP, T), lambda i: (i, 0, 0)),
        out_shape=jax.ShapeDtypeStruct((B * L, REP, T), jnp.float32),
    )(loc3)


def kernel(bank, loc_map, labels, sel_idx):
    bank_flat = bank.reshape(C * NQ, RD)
    flat_idx = (labels.astype(jnp.int32) * NQ)[:, :, None] + sel_idx.astype(jnp.int32)
    flat_idx = flat_idx.reshape(NROWS)
    hi = flat_idx[:SPLIT].reshape(16, BPW_HI)
    lo = jnp.pad(flat_idx[SPLIT:].reshape(16, BPW_LO),
                 ((0, 0), (0, BPW_HI - BPW_LO)))
    idx1d = jnp.concatenate([hi, lo]).reshape(NW * BPW_HI)

    rows = _sc_gather_rows(bank_flat, idx1d)
    batched_queries = rows.reshape(B, L * NQSEL * NS, D)

    mask = _tc_mask(loc_map.reshape(B * L, 1, T))
    batched_mask = mask.reshape(B, L * REP, T)

    batched_has_vision_query = jnp.ones((B, L), dtype=jnp.int32)
    return batched_queries, batched_mask, batched_has_vision_query


# speedup vs baseline: 9.5727x; 4.0849x over previous
import functools

import jax
import jax.numpy as jnp
from jax import lax
from jax.experimental import pallas as pl
from jax.experimental.pallas import tpu as pltpu
from jax.experimental.pallas import tpu_sc as plsc

B, L, T = 8, 80, 256
C, NQ, NS, D = 365, 100, 4, 256
NQSEL = 5
RD = NS * D
NROWS = B * L * NQSEL
NW = 32
BPW_HI = 104
BPW_LO = 96
SPLIT = 16 * BPW_HI
REP = NQSEL * NS


def _sc_gather_rows(bank3, idx1d):
    mesh = plsc.VectorSubcoreMesh(core_axis_name="c", subcore_axis_name="s")

    @functools.partial(
        pl.kernel,
        mesh=mesh,
        out_type=jax.ShapeDtypeStruct((NROWS * NS, D), jnp.float32),
        scratch_types=[
            pltpu.VMEM((BPW_HI,), jnp.int32),
            pltpu.VMEM((BPW_HI, NS, D), jnp.float32),
            pltpu.SemaphoreType.DMA,
        ],
    )
    def k(table_hbm, idx_hbm, out_hbm, idx_v, rows_v, sem):
        wid = lax.axis_index("s") * 2 + lax.axis_index("c")
        pltpu.sync_copy(idx_hbm.at[pl.ds(pl.multiple_of(wid * BPW_HI, 8),
                                         BPW_HI)], idx_v)
        pltpu.async_copy(table_hbm.at[idx_v], rows_v, sem).wait()
        flat = rows_v.reshape(BPW_HI * NS, D)

        @pl.when(wid < 16)
        def _hi():
            base = pl.multiple_of(wid * BPW_HI * NS, 8)
            pltpu.sync_copy(flat, out_hbm.at[pl.ds(base, BPW_HI * NS)])

        @pl.when(wid >= 16)
        def _lo():
            base = pl.multiple_of((SPLIT + (wid - 16) * BPW_LO) * NS, 8)
            pltpu.sync_copy(flat.at[pl.ds(0, BPW_LO * NS)],
                            out_hbm.at[pl.ds(base, BPW_LO * NS)])

    return k(bank3, idx1d)


def _tc_mask_body(loc_ref, out_ref):
    x = loc_ref[...]
    y = (x != 0.0).astype(jnp.float32)
    out_ref[...] = jnp.broadcast_to(y, out_ref.shape)


def _tc_mask(loc3):
    blk = 16
    grid = (B * L // blk,)
    return pl.pallas_call(
        _tc_mask_body,
        grid=grid,
        in_specs=[pl.BlockSpec((blk, 1, T), lambda i: (i, 0, 0))],
        out_specs=pl.BlockSpec((blk, REP, T), lambda i: (i, 0, 0)),
        out_shape=jax.ShapeDtypeStruct((B * L, REP, T), jnp.float32),
    )(loc3)


def kernel(bank, loc_map, labels, sel_idx):
    bank3 = bank.reshape(C * NQ, NS, D)
    flat_idx = (labels.astype(jnp.int32) * NQ)[:, :, None] + sel_idx.astype(jnp.int32)
    flat_idx = flat_idx.reshape(NROWS)
    hi = flat_idx[:SPLIT].reshape(16, BPW_HI)
    lo = jnp.pad(flat_idx[SPLIT:].reshape(16, BPW_LO),
                 ((0, 0), (0, BPW_HI - BPW_LO)))
    idx1d = jnp.concatenate([hi, lo]).reshape(NW * BPW_HI)

    rows = _sc_gather_rows(bank3, idx1d)
    batched_queries = rows.reshape(B, L * NQSEL * NS, D)

    mask = _tc_mask(loc_map.reshape(B * L, 1, T))
    batched_mask = mask.reshape(B, L * REP, T)

    batched_has_vision_query = jnp.ones((B, L), dtype=jnp.int32)
    return batched_queries, batched_mask, batched_has_vision_query


# speedup vs baseline: 12.3996x; 1.2953x over previous
import functools

import jax
import jax.numpy as jnp
from jax import lax
from jax.experimental import pallas as pl
from jax.experimental.pallas import tpu as pltpu
from jax.experimental.pallas import tpu_sc as plsc

B, L, T = 8, 80, 256
C, NQ, NS, D = 365, 100, 4, 256
NQSEL = 5
RD = NS * D
NROWS = B * L * NQSEL
NW = 32
BPW_HI = 104
BPW_LO = 96
SPLIT = 16 * BPW_HI
REP = NQSEL * NS


def _sc_gather_rows(bank3, idx1d):
    mesh = plsc.VectorSubcoreMesh(core_axis_name="c", subcore_axis_name="s")

    @functools.partial(
        pl.kernel,
        mesh=mesh,
        out_type=jax.ShapeDtypeStruct((NROWS * NS, D), jnp.float32),
        scratch_types=[
            pltpu.VMEM((BPW_HI,), jnp.int32),
            pltpu.VMEM((BPW_HI, NS, D), jnp.float32),
            pltpu.SemaphoreType.DMA,
        ],
    )
    def k(table_hbm, idx_hbm, out_hbm, idx_v, rows_v, sem):
        wid = lax.axis_index("s") * 2 + lax.axis_index("c")
        pltpu.sync_copy(idx_hbm.at[pl.ds(pl.multiple_of(wid * BPW_HI, 8),
                                         BPW_HI)], idx_v)
        pltpu.async_copy(table_hbm.at[idx_v], rows_v, sem).wait()
        flat = rows_v.reshape(BPW_HI * NS, D)

        @pl.when(wid < 16)
        def _hi():
            base = pl.multiple_of(wid * BPW_HI * NS, 8)
            pltpu.sync_copy(flat, out_hbm.at[pl.ds(base, BPW_HI * NS)])

        @pl.when(wid >= 16)
        def _lo():
            base = pl.multiple_of((SPLIT + (wid - 16) * BPW_LO) * NS, 8)
            pltpu.sync_copy(flat.at[pl.ds(0, BPW_LO * NS)],
                            out_hbm.at[pl.ds(base, BPW_LO * NS)])

    return k(bank3, idx1d)


MBLK = 16


def _tc_mask_body(loc_ref, out_ref):
    for j in range(MBLK):
        y = (loc_ref[j] != 0.0).astype(jnp.float32)
        out_ref[pl.ds(j * REP, REP), :] = jnp.broadcast_to(y, (REP, T))


def _tc_mask(loc3):
    grid = (B * L // MBLK,)
    return pl.pallas_call(
        _tc_mask_body,
        grid=grid,
        in_specs=[pl.BlockSpec((MBLK, 1, T), lambda i: (i, 0, 0))],
        out_specs=pl.BlockSpec((MBLK * REP, T), lambda i: (i, 0)),
        out_shape=jax.ShapeDtypeStruct((B * L * REP, T), jnp.float32),
    )(loc3)


def kernel(bank, loc_map, labels, sel_idx):
    bank3 = bank.reshape(C * NQ, NS, D)
    flat_idx = (labels.astype(jnp.int32) * NQ)[:, :, None] + sel_idx.astype(jnp.int32)
    flat_idx = flat_idx.reshape(NROWS)
    hi = flat_idx[:SPLIT].reshape(16, BPW_HI)
    lo = jnp.pad(flat_idx[SPLIT:].reshape(16, BPW_LO),
                 ((0, 0), (0, BPW_HI - BPW_LO)))
    idx1d = jnp.concatenate([hi, lo]).reshape(NW * BPW_HI)

    rows = _sc_gather_rows(bank3, idx1d)
    batched_queries = rows.reshape(B, L * NQSEL * NS, D)

    batched_mask = _tc_mask(loc_map.reshape(B * L, 1, T)).reshape(B, L * REP, T)

    batched_has_vision_query = jnp.ones((B, L), dtype=jnp.int32)
    return batched_queries, batched_mask, batched_has_vision_query
